# Initial kernel scaffold; baseline (speedup 1.0000x reference)
import jax
import jax.numpy as jnp
from jax.experimental import pallas as pl


def kernel(input_feature, input_lengths, vq_indices):
    raise NotImplementedError("write your pallas kernel here")


# speedup vs baseline: 3.4788x; 3.4788x over previous
import functools

import jax
import jax.numpy as jnp
from jax.experimental import pallas as pl
from jax.experimental.pallas import tpu as pltpu

_CHUNK = 512


def _weights_row(c0, c1, length, L):
    idx = jax.lax.broadcasted_iota(jnp.int32, (1, L), 1)
    valid = idx < length
    same = (c0 == pltpu.roll(c0, 1, axis=1)) & (c1 == pltpu.roll(c1, 1, axis=1))
    ng = ((idx == 0) | jnp.logical_not(same)) & valid

    s = jnp.where(ng, idx, -1)
    k = 1
    while k < L:
        s = jnp.maximum(s, jnp.where(idx >= k, pltpu.roll(s, k, axis=1), -1))
        k *= 2
    t = jnp.where(ng, idx, L)
    t = jnp.where(idx < L - 1, pltpu.roll(t, L - 1, axis=1), L)
    k = 1
    while k < L:
        t = jnp.minimum(t, jnp.where(idx < L - k, pltpu.roll(t, L - k, axis=1), L))
        k *= 2

    run_len = (jnp.minimum(t, length) - s).astype(jnp.float32)
    num_groups = jnp.sum(ng.astype(jnp.float32))
    denom = num_groups * run_len
    safe = valid & (denom > 0.0)
    return jnp.where(safe, 1.0 / jnp.where(safe, denom, 1.0), 0.0)


def _pool_kernel(len_ref, vq_ref, feat_ref, out_ref, w_ref, *, L, chunk):
    b = pl.program_id(0)
    lc = pl.program_id(1)

    @pl.when(lc == 0)
    def _():
        c0 = vq_ref[0, 0:1, :]
        c1 = vq_ref[0, 1:2, :]
        w_ref[...] = _weights_row(c0, c1, len_ref[b], L)

    w_chunk = w_ref[:, pl.ds(lc * chunk, chunk)]
    part = jnp.dot(w_chunk, feat_ref[0, 0],
                   preferred_element_type=jnp.float32)

    @pl.when(lc == 0)
    def _():
        out_ref[0] = part

    @pl.when(lc != 0)
    def _():
        out_ref[0] += part


@jax.jit
def kernel(input_feature, input_lengths, vq_indices):
    B, N, L, D = input_feature.shape
    lengths = input_lengths.astype(jnp.int32)
    vq_t = jnp.transpose(vq_indices.astype(jnp.int32), (0, 2, 1))

    n_chunks = L // _CHUNK
    grid_spec = pltpu.PrefetchScalarGridSpec(
        num_scalar_prefetch=1,
        grid=(B, n_chunks),
        in_specs=[
            pl.BlockSpec((1, 2, L), lambda b, lc, lens: (b, 0, 0)),
            pl.BlockSpec((1, 1, _CHUNK, D), lambda b, lc, lens: (b, N - 1, lc, 0)),
        ],
        out_specs=pl.BlockSpec((1, 1, D), lambda b, lc, lens: (b, 0, 0)),
        scratch_shapes=[pltpu.VMEM((1, L), jnp.float32)],
    )
    out = pl.pallas_call(
        functools.partial(_pool_kernel, L=L, chunk=_CHUNK),
        grid_spec=grid_spec,
        out_shape=jax.ShapeDtypeStruct((B, 1, D), jnp.float32),
    )(lengths, vq_t, input_feature)
    return out[:, 0, :]


# speedup vs baseline: 4.0321x; 1.1591x over previous
import functools

import jax
import jax.numpy as jnp
from jax.experimental import pallas as pl
from jax.experimental.pallas import tpu as pltpu

_CHUNK = 512


def _weights_all(c0, c1, lengths, L):
    B = c0.shape[0]
    idx = jax.lax.broadcasted_iota(jnp.int32, (B, L), 1)
    valid = idx < lengths
    same = (c0 == pltpu.roll(c0, 1, axis=1)) & (c1 == pltpu.roll(c1, 1, axis=1))
    ng = ((idx == 0) | jnp.logical_not(same)) & valid

    s = jnp.where(ng, idx, -1)
    k = 1
    while k < L:
        s = jnp.maximum(s, jnp.where(idx >= k, pltpu.roll(s, k, axis=1), -1))
        k *= 2
    t = jnp.where(ng, idx, L)
    t = jnp.where(idx < L - 1, pltpu.roll(t, L - 1, axis=1), L)
    k = 1
    while k < L:
        t = jnp.minimum(t, jnp.where(idx < L - k, pltpu.roll(t, L - k, axis=1), L))
        k *= 2

    run_len = (jnp.minimum(t, lengths) - s).astype(jnp.float32)
    num_groups = jnp.sum(ng.astype(jnp.float32), axis=1, keepdims=True)
    denom = num_groups * run_len
    safe = valid & (denom > 0.0)
    return jnp.where(safe, 1.0 / jnp.where(safe, denom, 1.0), 0.0)


def _pool_kernel(len_ref, vq_ref, feat_ref, out_ref, w_ref, *, B, L, chunk):
    b = pl.program_id(0)
    lc = pl.program_id(1)

    @pl.when((b == 0) & (lc == 0))
    def _():
        c0 = vq_ref[:, 0, :]
        c1 = vq_ref[:, 1, :]
        lengths = jnp.concatenate(
            [jnp.full((1, 1), len_ref[i], jnp.int32) for i in range(B)], axis=0)
        w_ref[...] = _weights_all(c0, c1, lengths, L)

    w_chunk = w_ref[pl.ds(b, 1), pl.ds(lc * chunk, chunk)]
    part = jnp.dot(w_chunk, feat_ref[0, 0],
                   preferred_element_type=jnp.float32)

    @pl.when(lc == 0)
    def _():
        out_ref[0] = part

    @pl.when(lc != 0)
    def _():
        out_ref[0] += part


@jax.jit
def kernel(input_feature, input_lengths, vq_indices):
    B, N, L, D = input_feature.shape
    lengths = input_lengths.astype(jnp.int32)
    vq_t = jnp.transpose(vq_indices.astype(jnp.int32), (0, 2, 1))

    n_chunks = L // _CHUNK
    grid_spec = pltpu.PrefetchScalarGridSpec(
        num_scalar_prefetch=1,
        grid=(B, n_chunks),
        in_specs=[
            pl.BlockSpec((B, 2, L), lambda b, lc, lens: (0, 0, 0)),
            pl.BlockSpec((1, 1, _CHUNK, D), lambda b, lc, lens: (b, N - 1, lc, 0)),
        ],
        out_specs=pl.BlockSpec((1, 1, D), lambda b, lc, lens: (b, 0, 0)),
        scratch_shapes=[pltpu.VMEM((B, L), jnp.float32)],
    )
    out = pl.pallas_call(
        functools.partial(_pool_kernel, B=B, L=L, chunk=_CHUNK),
        grid_spec=grid_spec,
        out_shape=jax.ShapeDtypeStruct((B, 1, D), jnp.float32),
    )(lengths, vq_t, input_feature)
    return out[:, 0, :]


# speedup vs baseline: 5.3654x; 1.3307x over previous
import functools

import jax
import jax.numpy as jnp
from jax.experimental import pallas as pl
from jax.experimental.pallas import tpu as pltpu

_CHUNK = 1024


def _weights_all(c0, c1, lengths, L):
    B = c0.shape[0]
    idx = jax.lax.broadcasted_iota(jnp.int32, (B, L), 1)
    valid = idx < lengths
    same = (c0 == pltpu.roll(c0, 1, axis=1)) & (c1 == pltpu.roll(c1, 1, axis=1))
    ng = ((idx == 0) | jnp.logical_not(same)) & valid

    s = jnp.where(ng, idx, -1)
    k = 1
    while k < L:
        s = jnp.maximum(s, jnp.where(idx >= k, pltpu.roll(s, k, axis=1), -1))
        k *= 2
    t = jnp.where(ng, idx, L)
    t = jnp.where(idx < L - 1, pltpu.roll(t, L - 1, axis=1), L)
    k = 1
    while k < L:
        t = jnp.minimum(t, jnp.where(idx < L - k, pltpu.roll(t, L - k, axis=1), L))
        k *= 2

    run_len = (jnp.minimum(t, lengths) - s).astype(jnp.float32)
    num_groups = jnp.sum(ng.astype(jnp.float32), axis=1, keepdims=True)
    denom = num_groups * run_len
    safe = valid & (denom > 0.0)
    return jnp.where(safe, 1.0 / jnp.where(safe, denom, 1.0), 0.0)


def _pool_kernel(len_ref, vq_ref, feat_ref, out_ref, w_ref, *, B, L, chunk):
    b = pl.program_id(0)
    lc = pl.program_id(1)

    @pl.when((b == 0) & (lc == 0))
    def _():
        c0 = vq_ref[:, 0, :]
        c1 = vq_ref[:, 1, :]
        lengths = jnp.concatenate(
            [jnp.full((1, 1), len_ref[i], jnp.int32) for i in range(B)], axis=0)
        w_ref[...] = _weights_all(c0, c1, lengths, L)

    w_chunk = w_ref[pl.ds(b, 1), pl.ds(lc * chunk, chunk)]
    part = jnp.dot(w_chunk, feat_ref[0, 0],
                   preferred_element_type=jnp.float32)

    @pl.when(lc == 0)
    def _():
        out_ref[0] = part

    @pl.when(lc != 0)
    def _():
        out_ref[0] += part


@jax.jit
def kernel(input_feature, input_lengths, vq_indices):
    B, N, L, D = input_feature.shape
    lengths = input_lengths.astype(jnp.int32)
    vq_t = jnp.transpose(vq_indices.astype(jnp.int32), (0, 2, 1))

    n_chunks = L // _CHUNK
    grid_spec = pltpu.PrefetchScalarGridSpec(
        num_scalar_prefetch=1,
        grid=(B, n_chunks),
        in_specs=[
            pl.BlockSpec((B, 2, L), lambda b, lc, lens: (0, 0, 0)),
            pl.BlockSpec((1, 1, _CHUNK, D), lambda b, lc, lens: (b, N - 1, lc, 0)),
        ],
        out_specs=pl.BlockSpec((1, 1, D), lambda b, lc, lens: (b, 0, 0)),
        scratch_shapes=[pltpu.VMEM((B, L), jnp.float32)],
    )
    out = pl.pallas_call(
        functools.partial(_pool_kernel, B=B, L=L, chunk=_CHUNK),
        grid_spec=grid_spec,
        out_shape=jax.ShapeDtypeStruct((B, 1, D), jnp.float32),
    )(lengths, vq_t, input_feature)
    return out[:, 0, :]


# speedup vs baseline: 5.9493x; 1.1088x over previous
import functools

import jax
import jax.numpy as jnp
from jax.experimental import pallas as pl
from jax.experimental.pallas import tpu as pltpu

_CHUNK = 2048


def _weights_all(c0, c1, lengths, L):
    B = c0.shape[0]
    idx = jax.lax.broadcasted_iota(jnp.int32, (B, L), 1)
    valid = idx < lengths
    same = (c0 == pltpu.roll(c0, 1, axis=1)) & (c1 == pltpu.roll(c1, 1, axis=1))
    ng = ((idx == 0) | jnp.logical_not(same)) & valid

    s = jnp.where(ng, idx, -1)
    k = 1
    while k < L:
        s = jnp.maximum(s, jnp.where(idx >= k, pltpu.roll(s, k, axis=1), -1))
        k *= 2
    t = jnp.where(ng, idx, L)
    t = jnp.where(idx < L - 1, pltpu.roll(t, L - 1, axis=1), L)
    k = 1
    while k < L:
        t = jnp.minimum(t, jnp.where(idx < L - k, pltpu.roll(t, L - k, axis=1), L))
        k *= 2

    run_len = (jnp.minimum(t, lengths) - s).astype(jnp.float32)
    num_groups = jnp.sum(ng.astype(jnp.float32), axis=1, keepdims=True)
    denom = num_groups * run_len
    safe = valid & (denom > 0.0)
    return jnp.where(safe, 1.0 / jnp.where(safe, denom, 1.0), 0.0)


def _pool_kernel(len_ref, vq_ref, feat_ref, out_ref, w_ref, *, B, L, chunk):
    b = pl.program_id(0)
    lc = pl.program_id(1)

    @pl.when((b == 0) & (lc == 0))
    def _():
        c0 = vq_ref[:, 0, :]
        c1 = vq_ref[:, 1, :]
        lengths = jnp.concatenate(
            [jnp.full((1, 1), len_ref[i], jnp.int32) for i in range(B)], axis=0)
        w_ref[...] = _weights_all(c0, c1, lengths, L)

    w_chunk = w_ref[pl.ds(b, 1), pl.ds(lc * chunk, chunk)]
    part = jnp.dot(w_chunk, feat_ref[0, 0],
                   preferred_element_type=jnp.float32)

    @pl.when(lc == 0)
    def _():
        out_ref[0] = part

    @pl.when(lc != 0)
    def _():
        out_ref[0] += part


@jax.jit
def kernel(input_feature, input_lengths, vq_indices):
    B, N, L, D = input_feature.shape
    lengths = input_lengths.astype(jnp.int32)
    vq_t = jnp.transpose(vq_indices.astype(jnp.int32), (0, 2, 1))

    n_chunks = L // _CHUNK
    grid_spec = pltpu.PrefetchScalarGridSpec(
        num_scalar_prefetch=1,
        grid=(B, n_chunks),
        in_specs=[
            pl.BlockSpec((B, 2, L), lambda b, lc, lens: (0, 0, 0)),
            pl.BlockSpec((1, 1, _CHUNK, D), lambda b, lc, lens: (b, N - 1, lc, 0)),
        ],
        out_specs=pl.BlockSpec((1, 1, D), lambda b, lc, lens: (b, 0, 0)),
        scratch_shapes=[pltpu.VMEM((B, L), jnp.float32)],
    )
    out = pl.pallas_call(
        functools.partial(_pool_kernel, B=B, L=L, chunk=_CHUNK),
        grid_spec=grid_spec,
        out_shape=jax.ShapeDtypeStruct((B, 1, D), jnp.float32),
    )(lengths, vq_t, input_feature)
    return out[:, 0, :]
